# Initial kernel scaffold; baseline (speedup 1.0000x reference)
import jax
import jax.numpy as jnp
from jax.experimental import pallas as pl


def kernel(x, offset):
    raise NotImplementedError("write your pallas kernel here")


# speedup vs baseline: 23.0005x; 23.0005x over previous
import functools

import jax
import jax.numpy as jnp
from jax import lax
from jax.experimental import pallas as pl

_POOLING_FACTOR = 0.25


def _rmax(a):
    return jnp.max(jnp.max(a, axis=1, keepdims=True), axis=0, keepdims=True)


def _rmin(a):
    return jnp.min(jnp.min(a, axis=1, keepdims=True), axis=0, keepdims=True)


def _rsum(a):
    return jnp.sum(jnp.sum(a, axis=1, keepdims=True), axis=0, keepdims=True)


def _fps_kernel(xr, yr, zr, out_ref, *, b, rows, n, m):
    iot = (lax.broadcasted_iota(jnp.int32, (rows, 128), 0) * 128
           + lax.broadcasted_iota(jnp.int32, (rows, 128), 1))
    X = [xr[i] for i in range(b)]
    Y = [yr[i] for i in range(b)]
    Z = [zr[i] for i in range(b)]

    def body(i, carry):
        new_carry = []
        for bi in range(b):
            d, cx, cy, cz = carry[bi]
            out_ref[bi, pl.ds(i, 1), :] = jnp.concatenate([cx, cy, cz], axis=1)
            dd = (X[bi] - cx) ** 2 + (Y[bi] - cy) ** 2 + (Z[bi] - cz) ** 2
            d = jnp.minimum(d, dd)
            mx = _rmax(d)
            idx = _rmin(jnp.where(d == mx, iot, n))
            oh = iot == idx
            cx = _rsum(jnp.where(oh, X[bi], 0.0))
            cy = _rsum(jnp.where(oh, Y[bi], 0.0))
            cz = _rsum(jnp.where(oh, Z[bi], 0.0))
            new_carry.append((d, cx, cy, cz))
        return tuple(new_carry)

    init = tuple(
        (jnp.full((rows, 128), jnp.inf, dtype=jnp.float32),
         X[bi][0:1, 0:1], Y[bi][0:1, 0:1], Z[bi][0:1, 0:1])
        for bi in range(b))
    lax.fori_loop(0, m, body, init, unroll=False)


def kernel(x, offset):
    b = offset.shape[0]
    n = x.shape[0] // b
    m = int(n * _POOLING_FACTOR)
    rows = n // 128
    coords = x[:, :3]
    xr = coords[:, 0].reshape(b, rows, 128)
    yr = coords[:, 1].reshape(b, rows, 128)
    zr = coords[:, 2].reshape(b, rows, 128)
    out = pl.pallas_call(
        functools.partial(_fps_kernel, b=b, rows=rows, n=n, m=m),
        out_shape=jax.ShapeDtypeStruct((b, m, 3), jnp.float32),
    )(xr, yr, zr)
    return out.reshape(b * m, 3)


# speedup vs baseline: 28.9378x; 1.2581x over previous
import functools

import jax
import jax.numpy as jnp
from jax import lax
from jax.experimental import pallas as pl

_POOLING_FACTOR = 0.25


def _rmax(a):
    return jnp.max(jnp.max(a, axis=0, keepdims=True), axis=1, keepdims=True)


def _rmin(a):
    return jnp.min(jnp.min(a, axis=0, keepdims=True), axis=1, keepdims=True)


def _rsum(a):
    return jnp.sum(jnp.sum(a, axis=0, keepdims=True), axis=1, keepdims=True)


def _fps_kernel(xr, yr, zr, out_ref, *, b, rows, n, m):
    iot = (lax.broadcasted_iota(jnp.int32, (rows, 128), 0) * 128
           + lax.broadcasted_iota(jnp.int32, (rows, 128), 1)
           ).astype(jnp.float32)
    big = jnp.float32(n)
    X = [xr[i] for i in range(b)]
    Y = [yr[i] for i in range(b)]
    Z = [zr[i] for i in range(b)]

    def body(i, carry):
        new_carry = []
        for bi in range(b):
            d, cx, cy, cz = carry[bi]
            out_ref[bi, pl.ds(i, 1), :] = jnp.concatenate([cx, cy, cz], axis=1)
            dd = (X[bi] - cx) ** 2 + (Y[bi] - cy) ** 2 + (Z[bi] - cz) ** 2
            d = jnp.minimum(d, dd)
            mx = _rmax(d)
            idx = _rmin(jnp.where(d == mx, iot, big))
            oh = iot == idx
            cx = _rsum(jnp.where(oh, X[bi], 0.0))
            cy = _rsum(jnp.where(oh, Y[bi], 0.0))
            cz = _rsum(jnp.where(oh, Z[bi], 0.0))
            new_carry.append((d, cx, cy, cz))
        return tuple(new_carry)

    init = tuple(
        (jnp.full((rows, 128), jnp.inf, dtype=jnp.float32),
         X[bi][0:1, 0:1], Y[bi][0:1, 0:1], Z[bi][0:1, 0:1])
        for bi in range(b))
    lax.fori_loop(0, m, body, init, unroll=False)


def kernel(x, offset):
    b = offset.shape[0]
    n = x.shape[0] // b
    m = int(n * _POOLING_FACTOR)
    rows = n // 128
    coords = x[:, :3]
    xr = coords[:, 0].reshape(b, rows, 128)
    yr = coords[:, 1].reshape(b, rows, 128)
    zr = coords[:, 2].reshape(b, rows, 128)
    out = pl.pallas_call(
        functools.partial(_fps_kernel, b=b, rows=rows, n=n, m=m),
        out_shape=jax.ShapeDtypeStruct((b, m, 3), jnp.float32),
    )(xr, yr, zr)
    return out.reshape(b * m, 3)
